# Initial kernel scaffold; baseline (speedup 1.0000x reference)
import jax
import jax.numpy as jnp
from jax.experimental import pallas as pl


def kernel(state, feature, edge_index, edge_feature, W):
    raise NotImplementedError("write your pallas kernel here")


# speedup vs baseline: 4.3114x; 4.3114x over previous
import functools

import jax
import jax.numpy as jnp
from jax import lax
from jax.experimental import pallas as pl
from jax.experimental.pallas import tpu as pltpu
from jax.experimental.pallas import tpu_sc as plsc

N = 10000
E = 320000
H = 128
HV = H // 16
NC = 2
NS = 16
NW = NC * NS
EW = E // NW
C = 80
NCHUNK = EW // C
RPS = 624
TAIL = N - NS * RPS


def _project_body(x_ref, w_ref, ps_ref, pd_ref):
    x = x_ref[...]
    w = w_ref[...]
    ps_ref[...] = jnp.dot(x, w[0:H, :], preferred_element_type=jnp.float32)
    pd_ref[...] = jnp.dot(x, w[H:2 * H, :], preferred_element_type=jnp.float32)


def _project(state, W):
    blk = 2000
    return pl.pallas_call(
        _project_body,
        grid=(N // blk,),
        in_specs=[pl.BlockSpec((blk, H), lambda i: (i, 0)),
                  pl.BlockSpec((2 * H, H), lambda i: (0, 0))],
        out_specs=[pl.BlockSpec((blk, H), lambda i: (i, 0)),
                   pl.BlockSpec((blk, H), lambda i: (i, 0))],
        out_shape=[jax.ShapeDtypeStruct((N, H), jnp.float32),
                   jax.ShapeDtypeStruct((N, H), jnp.float32)],
    )(state, W)


def _recip_body(a_ref, b_ref, o_ref):
    o_ref[...] = 1.0 / (a_ref[...] + b_ref[...] + 1e-16)


def _relu_body(a_ref, b_ref, o_ref):
    o_ref[...] = jnp.maximum(a_ref[...] + b_ref[...], 0.0)


def _elementwise(body, a, b):
    blk = 2000
    return pl.pallas_call(
        body,
        grid=(N // blk,),
        in_specs=[pl.BlockSpec((blk, H), lambda i: (i, 0)),
                  pl.BlockSpec((blk, H), lambda i: (i, 0))],
        out_specs=pl.BlockSpec((blk, H), lambda i: (i, 0)),
        out_shape=jax.ShapeDtypeStruct((N, H), jnp.float32),
    )(a, b)


def _zero_fill(buf):
    def zbody(i, carry):
        for j in range(HV):
            buf[i, pl.ds(j * 16, 16)] = jnp.zeros((16,), jnp.float32)
        return carry
    lax.fori_loop(0, C, zbody, 0)


def _acc_slices():
    out = []
    off = 0
    while off < RPS:
        sz = min(C, RPS - off)
        out.append((off, sz))
        off += sz
    return out


_SLICES = _acc_slices()


def _zero_shared(zbuf, acc_sh, sid):
    base = sid * RPS
    for off, sz in _SLICES:
        pltpu.sync_copy(zbuf.at[pl.ds(0, sz)], acc_sh.at[pl.ds(base + off, sz)])

    @pl.when(sid == 0)
    def _():
        pltpu.sync_copy(zbuf.at[pl.ds(0, TAIL)],
                        acc_sh.at[pl.ds(NS * RPS, TAIL)])


def _dump_shared(acc_sh, out_hbm, cid, sid):
    base = sid * RPS
    for off, sz in _SLICES:
        pltpu.sync_copy(acc_sh.at[pl.ds(base + off, sz)],
                        out_hbm.at[cid, pl.ds(base + off, sz)])

    @pl.when(sid == 0)
    def _():
        pltpu.sync_copy(acc_sh.at[pl.ds(NS * RPS, TAIL)],
                        out_hbm.at[cid, pl.ds(NS * RPS, TAIL)])


def _make_pass_a():
    mesh = plsc.VectorSubcoreMesh(core_axis_name="c", subcore_axis_name="s",
                                  num_cores=NC, num_subcores=NS)

    @functools.partial(
        pl.kernel,
        out_type=(jax.ShapeDtypeStruct((E, H), jnp.float32),
                  jax.ShapeDtypeStruct((NC, N, H), jnp.float32)),
        mesh=mesh,
        scratch_types=[
            pltpu.VMEM((C,), jnp.int32),
            pltpu.VMEM((C,), jnp.int32),
            pltpu.VMEM((C + 16,), jnp.float32),
            pltpu.VMEM((C, H), jnp.float32),
            pltpu.VMEM((C, H), jnp.float32),
            pltpu.VMEM_SHARED((N, H), jnp.float32),
            pltpu.SemaphoreType.DMA,
            pltpu.SemaphoreType.DMA,
        ],
    )
    def pass_a(src_hbm, dst_hbm, ef_hbm, ps_hbm, pd_hbm,
               w_hbm, dp_hbm,
               idx_s, idx_d, efv, rows_s, rows_d, den_sh, sem1, sem2):
        cid = lax.axis_index("c")
        sid = lax.axis_index("s")
        wid = sid * NC + cid

        _zero_fill(rows_s)
        _zero_shared(rows_s, den_sh, sid)
        plsc.subcore_barrier()

        ebase = wid * EW

        def chunk_body(k, carry):
            base = ebase + k * C
            pltpu.sync_copy(src_hbm.at[pl.ds(base, C)], idx_s)
            pltpu.sync_copy(dst_hbm.at[pl.ds(base, C)], idx_d)
            pltpu.sync_copy(ef_hbm.at[pl.ds(base, C)], efv.at[pl.ds(0, C)])
            cp1 = pltpu.async_copy(ps_hbm.at[idx_s], rows_s, sem1)
            cp2 = pltpu.async_copy(pd_hbm.at[idx_d], rows_d, sem2)
            cp1.wait()
            cp2.wait()

            def ebody(e, ecarry):
                ef = efv[pl.ds(e, 16)][0]
                for j in range(HV):
                    sl = pl.ds(j * 16, 16)
                    v = rows_s[e, sl] + rows_d[e, sl]
                    v = jnp.where(v >= 0.0, v, v * jnp.float32(0.01))
                    rows_s[e, sl] = jnp.exp(v * ef)
                return ecarry
            lax.fori_loop(0, C, ebody, 0)

            pltpu.sync_copy(rows_s, w_hbm.at[pl.ds(base, C)])
            pltpu.sync_copy(rows_s, den_sh.at[idx_d], add=True)
            return carry
        lax.fori_loop(0, NCHUNK, chunk_body, 0)

        plsc.subcore_barrier()
        _dump_shared(den_sh, dp_hbm, cid, sid)

    return pass_a


def _make_pass_b():
    mesh = plsc.VectorSubcoreMesh(core_axis_name="c", subcore_axis_name="s",
                                  num_cores=NC, num_subcores=NS)

    @functools.partial(
        pl.kernel,
        out_type=jax.ShapeDtypeStruct((NC, N, H), jnp.float32),
        mesh=mesh,
        scratch_types=[
            pltpu.VMEM((C,), jnp.int32),
            pltpu.VMEM((C,), jnp.int32),
            pltpu.VMEM((C, H), jnp.float32),
            pltpu.VMEM((C, H), jnp.float32),
            pltpu.VMEM((C, H), jnp.float32),
            pltpu.VMEM_SHARED((N, H), jnp.float32),
            pltpu.SemaphoreType.DMA,
            pltpu.SemaphoreType.DMA,
        ],
    )
    def pass_b(src_hbm, dst_hbm, w_hbm, st_hbm, rd_hbm,
               op_hbm,
               idx_s, idx_d, rows_w, rows_s, rows_d, out_sh, sem1, sem2):
        cid = lax.axis_index("c")
        sid = lax.axis_index("s")
        wid = sid * NC + cid

        _zero_fill(rows_w)
        _zero_shared(rows_w, out_sh, sid)
        plsc.subcore_barrier()

        ebase = wid * EW

        def chunk_body(k, carry):
            base = ebase + k * C
            pltpu.sync_copy(src_hbm.at[pl.ds(base, C)], idx_s)
            pltpu.sync_copy(dst_hbm.at[pl.ds(base, C)], idx_d)
            cpw = pltpu.async_copy(w_hbm.at[pl.ds(base, C)], rows_w, sem1)
            cps = pltpu.async_copy(st_hbm.at[idx_s], rows_s, sem2)
            cpw.wait()
            cpd = pltpu.async_copy(rd_hbm.at[idx_d], rows_d, sem1)
            cps.wait()
            cpd.wait()

            def ebody(e, ecarry):
                for j in range(HV):
                    sl = pl.ds(j * 16, 16)
                    rows_w[e, sl] = rows_w[e, sl] * rows_d[e, sl] * rows_s[e, sl]
                return ecarry
            lax.fori_loop(0, C, ebody, 0)

            pltpu.sync_copy(rows_w, out_sh.at[idx_d], add=True)
            return carry
        lax.fori_loop(0, NCHUNK, chunk_body, 0)

        plsc.subcore_barrier()
        _dump_shared(out_sh, op_hbm, cid, sid)

    return pass_b


_make_pass_a = functools.lru_cache(maxsize=None)(_make_pass_a)
_make_pass_b = functools.lru_cache(maxsize=None)(_make_pass_b)


def kernel(state, feature, edge_index, edge_feature, W):
    src = edge_index[0]
    dst = edge_index[1]
    ps, pd = _project(state, W)
    w, dp = _make_pass_a()(src, dst, edge_feature, ps, pd)
    rden = _elementwise(_recip_body, dp[0], dp[1])
    op = _make_pass_b()(src, dst, w, state, rden)
    return _elementwise(_relu_body, op[0], op[1])


# speedup vs baseline: 10.4277x; 2.4186x over previous
import functools

import jax
import jax.numpy as jnp
from jax import lax
from jax.experimental import pallas as pl
from jax.experimental.pallas import tpu as pltpu
from jax.experimental.pallas import tpu_sc as plsc

N = 10000
E = 320000
H = 128
HV = H // 16
NC = 2
NS = 16
NW = NC * NS
EW = E // NW
C = 56
NCHUNK = 179
EWP = NCHUNK * C
E2 = NW * EWP
NBUF = 3
NIDX = 6
RPS = 624
TAIL = N - NS * RPS


def _project_body(x_ref, w_ref, ps_ref, pd_ref):
    x = x_ref[...]
    w = w_ref[...]
    ps_ref[...] = jnp.dot(x, w[0:H, :], preferred_element_type=jnp.float32)
    pd_ref[...] = jnp.dot(x, w[H:2 * H, :], preferred_element_type=jnp.float32)


def _project(state, W):
    blk = 2000
    return pl.pallas_call(
        _project_body,
        grid=(N // blk,),
        in_specs=[pl.BlockSpec((blk, H), lambda i: (i, 0)),
                  pl.BlockSpec((2 * H, H), lambda i: (0, 0))],
        out_specs=[pl.BlockSpec((blk, H), lambda i: (i, 0)),
                   pl.BlockSpec((blk, H), lambda i: (i, 0))],
        out_shape=[jax.ShapeDtypeStruct((N, H), jnp.float32),
                   jax.ShapeDtypeStruct((N, H), jnp.float32)],
    )(state, W)


def _final_body(a_ref, b_ref, c_ref, d_ref, o_ref):
    num = a_ref[...] + b_ref[...]
    den = c_ref[...] + d_ref[...] + 1e-16
    o_ref[...] = jnp.maximum(num / den, 0.0)


def _final(op0, op1, dp0, dp1):
    blk = 2000
    spec = pl.BlockSpec((blk, H), lambda i: (i, 0))
    return pl.pallas_call(
        _final_body,
        grid=(N // blk,),
        in_specs=[spec, spec, spec, spec],
        out_specs=spec,
        out_shape=jax.ShapeDtypeStruct((N, H), jnp.float32),
    )(op0, op1, dp0, dp1)


def _zero_fill(buf):
    def zbody(i, carry):
        for j in range(HV):
            buf[i, pl.ds(j * 16, 16)] = jnp.zeros((16,), jnp.float32)
        return carry
    lax.fori_loop(0, C, zbody, 0)


def _acc_slices():
    out = []
    off = 0
    while off < RPS:
        sz = min(C, RPS - off)
        out.append((off, sz))
        off += sz
    return out


_SLICES = _acc_slices()


def _zero_shared(zbuf, acc_sh, sid):
    base = sid * RPS
    for off, sz in _SLICES:
        pltpu.sync_copy(zbuf.at[pl.ds(0, sz)], acc_sh.at[pl.ds(base + off, sz)])

    @pl.when(sid == 0)
    def _():
        pltpu.sync_copy(zbuf.at[pl.ds(0, TAIL)],
                        acc_sh.at[pl.ds(NS * RPS, TAIL)])


def _dump_shared(acc_sh, out_hbm, cid, sid):
    base = sid * RPS
    for off, sz in _SLICES:
        pltpu.sync_copy(acc_sh.at[pl.ds(base + off, sz)],
                        out_hbm.at[cid, pl.ds(base + off, sz)])

    @pl.when(sid == 0)
    def _():
        pltpu.sync_copy(acc_sh.at[pl.ds(NS * RPS, TAIL)],
                        out_hbm.at[cid, pl.ds(NS * RPS, TAIL)])


def _sc_mesh():
    return plsc.VectorSubcoreMesh(core_axis_name="c", subcore_axis_name="s",
                                  num_cores=NC, num_subcores=NS)


@functools.lru_cache(maxsize=None)
def _make_pass_a():
    scratch = ([pltpu.VMEM((C,), jnp.int32)] * NIDX
               + [pltpu.VMEM((C,), jnp.int32)] * NIDX
               + [pltpu.VMEM((C + 16,), jnp.float32)] * NIDX
               + [pltpu.VMEM((C, H), jnp.float32)] * (2 * NBUF)
               + [pltpu.VMEM_SHARED((N, H), jnp.float32)]
               + [pltpu.SemaphoreType.DMA] * (NIDX + 4 * NBUF))

    @functools.partial(
        pl.kernel,
        out_type=(jax.ShapeDtypeStruct((E2, H), jnp.float32),
                  jax.ShapeDtypeStruct((NC, N, H), jnp.float32)),
        mesh=_sc_mesh(),
        scratch_types=scratch,
    )
    def pass_a(src_hbm, dst_hbm, ef_hbm, ps_hbm, pd_hbm,
               w_hbm, dp_hbm, *sc):
        sidx = sc[0:NIDX]
        didx = sc[NIDX:2 * NIDX]
        efs = sc[2 * NIDX:3 * NIDX]
        rs = sc[3 * NIDX:3 * NIDX + NBUF]
        rd = sc[3 * NIDX + NBUF:3 * NIDX + 2 * NBUF]
        den_sh = sc[3 * NIDX + 2 * NBUF]
        sems = sc[3 * NIDX + 2 * NBUF + 1:]
        si = sems[0:NIDX]
        sg1, sg2 = sems[NIDX:NIDX + NBUF], sems[NIDX + NBUF:NIDX + 2 * NBUF]
        ssc = sems[NIDX + 2 * NBUF:NIDX + 3 * NBUF]
        sw = sems[NIDX + 3 * NBUF:NIDX + 4 * NBUF]

        cid = lax.axis_index("c")
        sid = lax.axis_index("s")
        wid = sid * NC + cid
        ebase = wid * EWP

        def idx_descs(k, q):
            base = ebase + k * C
            return (
                pltpu.make_async_copy(src_hbm.at[pl.ds(base, C)], sidx[q], si[q]),
                pltpu.make_async_copy(dst_hbm.at[pl.ds(base, C)], didx[q], si[q]),
                pltpu.make_async_copy(ef_hbm.at[pl.ds(base, C)],
                                      efs[q].at[pl.ds(0, C)], si[q]),
            )

        def issue_idx(k, q):
            for d in idx_descs(k, q):
                d.start()

        def wait_idx(k, q):
            for d in idx_descs(k, q):
                d.wait()

        def issue_in(k, b, q):
            pltpu.async_copy(ps_hbm.at[sidx[q]], rs[b], sg1[b])
            pltpu.async_copy(pd_hbm.at[didx[q]], rd[b], sg2[b])

        def wait_in(k, b, q):
            pltpu.make_async_copy(ps_hbm.at[sidx[q]], rs[b], sg1[b]).wait()
            pltpu.make_async_copy(pd_hbm.at[didx[q]], rd[b], sg2[b]).wait()

        def issue_out(k, b, q):
            base = ebase + k * C
            pltpu.async_copy(rs[b], w_hbm.at[pl.ds(base, C)], sw[b])
            pltpu.async_copy(rs[b], den_sh.at[didx[q]], ssc[b], add=True)

        def wait_out(k, b, q):
            base = ebase + k * C
            pltpu.make_async_copy(rs[b], w_hbm.at[pl.ds(base, C)], sw[b]).wait()
            pltpu.make_async_copy(rs[b], den_sh.at[didx[q]], ssc[b]).wait()

        def compute(k, b, q):
            def ebody(e, ecarry):
                kl = k * C + e
                ef = efs[q][pl.ds(e, 16)][0]
                m = (kl < EW).astype(jnp.float32)
                for j in range(HV):
                    sl = pl.ds(j * 16, 16)
                    v = rs[b][e, sl] + rd[b][e, sl]
                    v = jnp.where(v >= 0.0, v, v * jnp.float32(0.01))
                    rs[b][e, sl] = jnp.exp(v * ef) * m
                return ecarry
            lax.fori_loop(0, C, ebody, 0)

        _zero_fill(rs[0])
        _zero_shared(rs[0], den_sh, sid)
        issue_idx(0, 0)
        issue_idx(1, 1)
        wait_idx(0, 0)
        issue_in(0, 0, 0)
        plsc.subcore_barrier()

        ngroups = (NCHUNK + NIDX - 1) // NIDX

        def group(i, carry):
            for j in range(NIDX):
                k = i * NIDX + j
                b = j % NBUF
                q = j

                @pl.when(k < NCHUNK)
                def _():
                    nb = (j + 1) % NBUF
                    nq = (j + 1) % NIDX

                    @pl.when(k >= 2)
                    def _():
                        wait_out(k - 2, nb, (j + 4) % NIDX)

                    @pl.when(k + 1 < NCHUNK)
                    def _():
                        wait_idx(k + 1, nq)
                        issue_in(k + 1, nb, nq)

                    @pl.when(k + 2 < NCHUNK)
                    def _():
                        issue_idx(k + 2, (j + 2) % NIDX)

                    wait_in(k, b, q)
                    compute(k, b, q)
                    issue_out(k, b, q)
            return carry
        lax.fori_loop(0, ngroups, group, 0)

        wait_out(NCHUNK - 2, (NCHUNK - 2) % NBUF, (NCHUNK - 2) % NIDX)
        wait_out(NCHUNK - 1, (NCHUNK - 1) % NBUF, (NCHUNK - 1) % NIDX)
        plsc.subcore_barrier()
        _dump_shared(den_sh, dp_hbm, cid, sid)

    return pass_a


@functools.lru_cache(maxsize=None)
def _make_pass_b():
    scratch = ([pltpu.VMEM((C,), jnp.int32)] * NIDX
               + [pltpu.VMEM((C,), jnp.int32)] * NIDX
               + [pltpu.VMEM((C, H), jnp.float32)] * (2 * NBUF)
               + [pltpu.VMEM_SHARED((N, H), jnp.float32)]
               + [pltpu.SemaphoreType.DMA] * (NIDX + 3 * NBUF))

    @functools.partial(
        pl.kernel,
        out_type=jax.ShapeDtypeStruct((NC, N, H), jnp.float32),
        mesh=_sc_mesh(),
        scratch_types=scratch,
    )
    def pass_b(src_hbm, dst_hbm, w_hbm, st_hbm,
               op_hbm, *sc):
        sidx = sc[0:NIDX]
        didx = sc[NIDX:2 * NIDX]
        rw = sc[2 * NIDX:2 * NIDX + NBUF]
        rs = sc[2 * NIDX + NBUF:2 * NIDX + 2 * NBUF]
        out_sh = sc[2 * NIDX + 2 * NBUF]
        sems = sc[2 * NIDX + 2 * NBUF + 1:]
        si = sems[0:NIDX]
        sgw, sgs = sems[NIDX:NIDX + NBUF], sems[NIDX + NBUF:NIDX + 2 * NBUF]
        ssc = sems[NIDX + 2 * NBUF:NIDX + 3 * NBUF]

        cid = lax.axis_index("c")
        sid = lax.axis_index("s")
        wid = sid * NC + cid
        ebase = wid * EWP

        def idx_descs(k, q):
            base = ebase + k * C
            return (
                pltpu.make_async_copy(src_hbm.at[pl.ds(base, C)], sidx[q], si[q]),
                pltpu.make_async_copy(dst_hbm.at[pl.ds(base, C)], didx[q], si[q]),
            )

        def issue_idx(k, q):
            for d in idx_descs(k, q):
                d.start()

        def wait_idx(k, q):
            for d in idx_descs(k, q):
                d.wait()

        def issue_in(k, b, q):
            base = ebase + k * C
            pltpu.async_copy(w_hbm.at[pl.ds(base, C)], rw[b], sgw[b])
            pltpu.async_copy(st_hbm.at[sidx[q]], rs[b], sgs[b])

        def wait_in(k, b, q):
            base = ebase + k * C
            pltpu.make_async_copy(w_hbm.at[pl.ds(base, C)], rw[b], sgw[b]).wait()
            pltpu.make_async_copy(st_hbm.at[sidx[q]], rs[b], sgs[b]).wait()

        def issue_out(k, b, q):
            pltpu.async_copy(rw[b], out_sh.at[didx[q]], ssc[b], add=True)

        def wait_out(k, b, q):
            pltpu.make_async_copy(rw[b], out_sh.at[didx[q]], ssc[b]).wait()

        def compute(k, b, q):
            def ebody(e, ecarry):
                for j in range(HV):
                    sl = pl.ds(j * 16, 16)
                    rw[b][e, sl] = rw[b][e, sl] * rs[b][e, sl]
                return ecarry
            lax.fori_loop(0, C, ebody, 0)

        _zero_fill(rw[0])
        _zero_shared(rw[0], out_sh, sid)
        issue_idx(0, 0)
        issue_idx(1, 1)
        wait_idx(0, 0)
        issue_in(0, 0, 0)
        plsc.subcore_barrier()

        ngroups = (NCHUNK + NIDX - 1) // NIDX

        def group(i, carry):
            for j in range(NIDX):
                k = i * NIDX + j
                b = j % NBUF
                q = j

                @pl.when(k < NCHUNK)
                def _():
                    nb = (j + 1) % NBUF
                    nq = (j + 1) % NIDX

                    @pl.when(k >= 2)
                    def _():
                        wait_out(k - 2, nb, (j + 4) % NIDX)

                    @pl.when(k + 1 < NCHUNK)
                    def _():
                        wait_idx(k + 1, nq)
                        issue_in(k + 1, nb, nq)

                    @pl.when(k + 2 < NCHUNK)
                    def _():
                        issue_idx(k + 2, (j + 2) % NIDX)

                    wait_in(k, b, q)
                    compute(k, b, q)
                    issue_out(k, b, q)
            return carry
        lax.fori_loop(0, ngroups, group, 0)

        wait_out(NCHUNK - 2, (NCHUNK - 2) % NBUF, (NCHUNK - 2) % NIDX)
        wait_out(NCHUNK - 1, (NCHUNK - 1) % NBUF, (NCHUNK - 1) % NIDX)
        plsc.subcore_barrier()
        _dump_shared(out_sh, op_hbm, cid, sid)

    return pass_b


def _pad_edges(x):
    return jnp.concatenate(
        [x.reshape(NW, EW),
         jnp.zeros((NW, EWP - EW), x.dtype)], axis=1).reshape(-1)


def kernel(state, feature, edge_index, edge_feature, W):
    srcp = _pad_edges(edge_index[0])
    dstp = _pad_edges(edge_index[1])
    efp = _pad_edges(edge_feature)
    ps, pd = _project(state, W)
    w, dp = _make_pass_a()(srcp, dstp, efp, ps, pd)
    op = _make_pass_b()(srcp, dstp, w, state)
    return _final(op[0], op[1], dp[0], dp[1])
